# Initial kernel scaffold; baseline (speedup 1.0000x reference)
import jax
import jax.numpy as jnp
from jax.experimental import pallas as pl


def kernel(inputs, table, W, b):
    raise NotImplementedError("write your pallas kernel here")


# speedup vs baseline: 4.9796x; 4.9796x over previous
import functools

import jax
import jax.numpy as jnp
from jax import lax
from jax.experimental import pallas as pl
from jax.experimental.pallas import tpu as pltpu
from jax.experimental.pallas import tpu_sc as plsc

_D = 32
_OUT = 350
_NW = 32
_ROW = 128
_CH_ROWS = 16


@functools.partial(jax.jit, static_argnums=(2,))
def _gather(idx3, table, n_total):
    per_w = n_total // _NW
    jr = per_w // _ROW
    nch = jr // _CH_ROWS
    chunk = _CH_ROWS * _ROW

    mesh = plsc.VectorSubcoreMesh(core_axis_name="c", subcore_axis_name="s")

    @functools.partial(
        pl.kernel,
        mesh=mesh,
        out_type=jax.ShapeDtypeStruct((_NW, nch, chunk, _D), jnp.float32),
        scratch_types=[
            pltpu.VMEM((jr, _ROW), jnp.int32),
            pltpu.VMEM((chunk, _D), jnp.float32),
            pltpu.SemaphoreType.DMA,
        ],
        compiler_params=pltpu.CompilerParams(use_tc_tiling_on_sc=False),
    )
    def gather(idx_hbm, table_hbm, out_hbm, idx_v, rows_v, sem):
        wid = lax.axis_index("s") * 2 + lax.axis_index("c")
        pltpu.sync_copy(idx_hbm.at[wid], idx_v)

        def body(ch, carry):
            copies = []
            for t in range(_CH_ROWS):
                copies.append(
                    pltpu.async_copy(
                        table_hbm.at[idx_v.at[ch * _CH_ROWS + t]],
                        rows_v.at[pl.ds(t * _ROW, _ROW)],
                        sem,
                    )
                )
            for cp in copies:
                cp.wait()
            pltpu.sync_copy(rows_v, out_hbm.at[wid, ch])
            return carry

        lax.fori_loop(0, nch, body, 0)

    return gather(idx3, table)


def _mm_body(e_ref, w_ref, b_ref, o_ref):
    o_ref[...] = (
        jnp.dot(e_ref[...], w_ref[...], preferred_element_type=jnp.float32)
        + b_ref[...]
    )


_BLK = 2048


def _dense(emb, W, b2, n):
    return pl.pallas_call(
        _mm_body,
        grid=(n // _BLK,),
        in_specs=[
            pl.BlockSpec((_BLK, _D), lambda i: (i, 0)),
            pl.BlockSpec((_D, _OUT), lambda i: (0, 0)),
            pl.BlockSpec((1, _OUT), lambda i: (0, 0)),
        ],
        out_specs=pl.BlockSpec((_BLK, _OUT), lambda i: (i, 0)),
        out_shape=jax.ShapeDtypeStruct((n, _OUT), jnp.float32),
    )(emb, W, b2)


def kernel(inputs, table, W, b):
    B, L = inputs.shape
    n = B * L
    idx3 = inputs.reshape(_NW, n // _NW // _ROW, _ROW)
    emb = _gather(idx3, table, n).reshape(n, _D)
    out = _dense(emb, W, b.reshape(1, _OUT), n)
    return out.reshape(B, L, _OUT)
